# baseline (speedup 1.0000x reference)
import functools

import jax
import jax.numpy as jnp
from jax.experimental import pallas as pl
from jax.experimental.pallas import tpu as pltpu

_BI = 200


def _body(adja_ref, adjb_ref, x_ref, w1_ref, b1_ref, wc_ref, bc_ref,
          mu_ref, sig_ref, hp_ref, *, nlat, bi):
    p = pl.program_id(0)
    i = pl.program_id(1)
    a = adja_ref[...].astype(jnp.bfloat16)
    b = adjb_ref[...].astype(jnp.bfloat16)

    @pl.when(p == 0)
    def _phase0():
        def hp_block(blk):
            ax = jnp.dot(blk, x_ref[...], preferred_element_type=jnp.float32)
            h = jnp.dot(ax.astype(jnp.bfloat16), w1_ref[...],
                        preferred_element_type=jnp.float32)
            h = jnp.maximum(h + b1_ref[...], 0.0)
            hp = jnp.dot(h.astype(jnp.bfloat16), wc_ref[...],
                         preferred_element_type=jnp.float32)
            return hp.astype(jnp.bfloat16)

        hp_ref[pl.ds(2 * i * bi, bi), :] = hp_block(a)
        hp_ref[pl.ds((2 * i + 1) * bi, bi), :] = hp_block(b)

    @pl.when(p == 1)
    def _phase1():
        hp = hp_ref[...]
        oa = jnp.dot(a, hp, preferred_element_type=jnp.float32) + bc_ref[...]
        ob = jnp.dot(b, hp, preferred_element_type=jnp.float32) + bc_ref[...]
        mu_ref[:bi, :] = oa[:, :nlat]
        mu_ref[bi:, :] = ob[:, :nlat]
        sig_ref[:bi, :] = jnp.exp(oa[:, nlat:])
        sig_ref[bi:, :] = jnp.exp(ob[:, nlat:])


def kernel(x, adj, W1, b1, W_mu, b_mu, W_sig, b_sig):
    n, n_feat = x.shape
    n_hid = W1.shape[1]
    n_lat = W_mu.shape[1]
    bi = _BI

    x_b = x.astype(jnp.bfloat16)
    w1_b = W1.astype(jnp.bfloat16)
    wc_b = jnp.concatenate([W_mu, W_sig], axis=1).astype(jnp.bfloat16)
    b1_2d = b1.reshape(1, n_hid)
    bc_2d = jnp.concatenate([b_mu, b_sig]).reshape(1, 2 * n_lat)

    mu, sig = pl.pallas_call(
        functools.partial(_body, nlat=n_lat, bi=bi),
        grid=(2, n // (2 * bi)),
        in_specs=[
            pl.BlockSpec((bi, n), lambda p, i: (2 * i, 0)),
            pl.BlockSpec((bi, n), lambda p, i: (2 * i + 1, 0)),
            pl.BlockSpec((n, n_feat), lambda p, i: (0, 0)),
            pl.BlockSpec((n_feat, n_hid), lambda p, i: (0, 0)),
            pl.BlockSpec((1, n_hid), lambda p, i: (0, 0)),
            pl.BlockSpec((n_hid, 2 * n_lat), lambda p, i: (0, 0)),
            pl.BlockSpec((1, 2 * n_lat), lambda p, i: (0, 0)),
        ],
        out_specs=[
            pl.BlockSpec((2 * bi, n_lat), lambda p, i: (i * p, 0)),
            pl.BlockSpec((2 * bi, n_lat), lambda p, i: (i * p, 0)),
        ],
        out_shape=[
            jax.ShapeDtypeStruct((n, n_lat), jnp.float32),
            jax.ShapeDtypeStruct((n, n_lat), jnp.float32),
        ],
        scratch_shapes=[pltpu.VMEM((n, 2 * n_lat), jnp.bfloat16)],
        compiler_params=pltpu.CompilerParams(
            dimension_semantics=("arbitrary", "arbitrary")),
    )(adj, adj, x_b, w1_b, b1_2d, wc_b, bc_2d)

    return (mu, sig)
